# baseline (speedup 1.0000x reference)
import functools

import jax
import jax.numpy as jnp
from jax import lax
from jax.experimental import pallas as pl
from jax.experimental.pallas import tpu as pltpu
from jax.experimental.pallas import tpu_sc as plsc

MAXLEN = 200
EMBED = 64
BATCH = 1024
NW = 32
ROWS_PER_W = BATCH // NW
CHUNK = 100
NCHUNK = MAXLEN // CHUNK
LANES = 16
CPR = EMBED // LANES


def _body(x_hbm, tok_hbm, pos_hbm, out_hbm,
          idx_v, gbuf0, gbuf1, obuf0, obuf1, pos_v,
          gsem0, gsem1, osem0, osem1):
    wid = lax.axis_index("s") * 2 + lax.axis_index("c")
    base = wid * ROWS_PER_W

    gbuf = (gbuf0, gbuf1)
    obuf = (obuf0, obuf1)
    gsem = (gsem0, gsem1)
    osem = (osem0, osem1)

    pltpu.sync_copy(x_hbm.at[pl.ds(base * NCHUNK, ROWS_PER_W * NCHUNK)], idx_v)
    pltpu.sync_copy(pos_hbm, pos_v)

    def start_gather(t):
        p = t % 2
        return [
            pltpu.async_copy(
                tok_hbm.at[idx_v.at[t * NCHUNK + j]],
                gbuf[p].at[pl.ds(j * CHUNK, CHUNK)],
                gsem[p],
            )
            for j in range(NCHUNK)
        ]

    def add_pos(src, dst):
        def row(r, carry):
            for c in range(CPR):
                s = pl.ds(c * LANES, LANES)
                dst[r, s] = src[r, s] + pos_v[r, s]
            return carry
        lax.fori_loop(0, MAXLEN, row, 0, unroll=4)

    gh = [start_gather(0), start_gather(1)]
    oh = [None, None]
    for t in range(ROWS_PER_W):
        p = t % 2
        for h in gh[p]:
            h.wait()
        add_pos(gbuf[p], obuf[p])
        if t + 2 < ROWS_PER_W:
            gh[p] = start_gather(t + 2)
        if oh[p] is not None:
            oh[p].wait()
        oh[p] = pltpu.async_copy(obuf[p], out_hbm.at[base + t], osem[p])
    for h in oh:
        if h is not None:
            h.wait()


@functools.partial(jax.jit, static_argnames=())
def _emb(x2, token_table, pos_table):
    mesh = plsc.VectorSubcoreMesh(core_axis_name="c", subcore_axis_name="s")
    run = pl.kernel(
        _body,
        out_type=jax.ShapeDtypeStruct((BATCH, MAXLEN, EMBED), jnp.float32),
        mesh=mesh,
        compiler_params=pltpu.CompilerParams(use_tc_tiling_on_sc=False),
        scratch_types=[
            pltpu.VMEM((ROWS_PER_W * NCHUNK, CHUNK), jnp.int32),
            pltpu.VMEM((MAXLEN, EMBED), jnp.float32),
            pltpu.VMEM((MAXLEN, EMBED), jnp.float32),
            pltpu.VMEM((MAXLEN, EMBED), jnp.float32),
            pltpu.VMEM((MAXLEN, EMBED), jnp.float32),
            pltpu.VMEM((MAXLEN, EMBED), jnp.float32),
            pltpu.SemaphoreType.DMA,
            pltpu.SemaphoreType.DMA,
            pltpu.SemaphoreType.DMA,
            pltpu.SemaphoreType.DMA,
        ],
    )
    return run(x2, token_table, pos_table)


def kernel(x, token_table, pos_table):
    x2 = x.astype(jnp.int32).reshape(BATCH * NCHUNK, CHUNK)
    return _emb(x2, token_table, pos_table)


# speedup vs baseline: 1.4645x; 1.4645x over previous
import jax
import jax.numpy as jnp
from jax import lax
from jax.experimental import pallas as pl
from jax.experimental.pallas import tpu as pltpu
from jax.experimental.pallas import tpu_sc as plsc

MAXLEN = 200
LPAD = 256
EMBED = 64
BATCH = 1024
NW = 32
ROWS_PER_W = BATCH // NW
NBUF = 3
LANES = 16
CPR = EMBED // LANES


def _body(x_hbm, tok_hbm, pos_hbm, out_hbm,
          xidx_v, pos_v, spmem_x, gbufs, sidxs, gsems, osems, isems):
    sid = lax.axis_index("s")
    wid = sid * 2 + lax.axis_index("c")
    base = wid * ROWS_PER_W

    pltpu.sync_copy(x_hbm.at[pl.ds(base, ROWS_PER_W)], xidx_v)
    pltpu.sync_copy(xidx_v, spmem_x.at[sid])
    pltpu.sync_copy(pos_hbm, pos_v)

    def stage_idx(t):
        p = t % NBUF
        return pltpu.async_copy(spmem_x.at[sid, t], sidxs[p], isems[p])

    def fire_gathers(t):
        p = t % NBUF
        sidx = sidxs[p]
        gbuf = gbufs[p]
        sem = gsems[p]

        def one(i, carry):
            idx = sidx[i]
            pltpu.async_copy(tok_hbm.at[idx], gbuf.at[i], sem)
            return carry
        lax.fori_loop(0, MAXLEN, one, 0, unroll=4)

    def drain_gathers(t):
        p = t % NBUF

        def one(i, carry):
            pltpu.make_async_copy(tok_hbm.at[0], gbufs[p].at[0], gsems[p]).wait()
            return carry
        lax.fori_loop(0, MAXLEN, one, 0, unroll=4)

    def add_pos(t):
        p = t % NBUF
        gbuf = gbufs[p]

        def row(r, carry):
            for c in range(CPR):
                s = pl.ds(c * LANES, LANES)
                gbuf[r, s] = gbuf[r, s] + pos_v[r, s]
            return carry
        lax.fori_loop(0, MAXLEN, row, 0, unroll=4)

    ih = [None] * NBUF
    oh = [None] * NBUF
    for t in range(min(NBUF, ROWS_PER_W)):
        ih[t % NBUF] = stage_idx(t)
    ih[0].wait()
    fire_gathers(0)

    for t in range(ROWS_PER_W):
        p = t % NBUF
        q = (t + 1) % NBUF
        if t + 1 < ROWS_PER_W:
            if oh[q] is not None:
                oh[q].wait()
                oh[q] = None
            ih[q].wait()
            fire_gathers(t + 1)
        if t + NBUF < ROWS_PER_W:
            ih[p] = stage_idx(t + NBUF)
        drain_gathers(t)
        add_pos(t)
        oh[p] = pltpu.async_copy(gbufs[p], out_hbm.at[base + t], osems[p])
    for h in oh:
        if h is not None:
            h.wait()


@jax.jit
def _emb(x, token_table, pos_table):
    mesh = plsc.VectorSubcoreMesh(core_axis_name="c", subcore_axis_name="s")

    def body(x_hbm, tok_hbm, pos_hbm, out_hbm,
             xidx_v, pos_v, spmem_x,
             g0, g1, g2, s0, s1, s2,
             gs0, gs1, gs2, os0, os1, os2, is0, is1, is2):
        _body(x_hbm, tok_hbm, pos_hbm, out_hbm, xidx_v, pos_v, spmem_x,
              (g0, g1, g2), (s0, s1, s2),
              (gs0, gs1, gs2), (os0, os1, os2), (is0, is1, is2))

    run = pl.kernel(
        body,
        out_type=jax.ShapeDtypeStruct((BATCH, MAXLEN, EMBED), jnp.float32),
        mesh=mesh,
        compiler_params=pltpu.CompilerParams(use_tc_tiling_on_sc=True),
        scratch_types=[
            pltpu.VMEM((ROWS_PER_W, LPAD), jnp.int32),
            pltpu.VMEM((MAXLEN, EMBED), jnp.float32),
            pltpu.VMEM_SHARED((16, ROWS_PER_W, LPAD), jnp.int32),
            pltpu.VMEM((MAXLEN, EMBED), jnp.float32),
            pltpu.VMEM((MAXLEN, EMBED), jnp.float32),
            pltpu.VMEM((MAXLEN, EMBED), jnp.float32),
            pltpu.SMEM((LPAD,), jnp.int32),
            pltpu.SMEM((LPAD,), jnp.int32),
            pltpu.SMEM((LPAD,), jnp.int32),
            pltpu.SemaphoreType.DMA,
            pltpu.SemaphoreType.DMA,
            pltpu.SemaphoreType.DMA,
            pltpu.SemaphoreType.DMA,
            pltpu.SemaphoreType.DMA,
            pltpu.SemaphoreType.DMA,
            pltpu.SemaphoreType.DMA,
            pltpu.SemaphoreType.DMA,
            pltpu.SemaphoreType.DMA,
        ],
    )
    return run(x, token_table, pos_table)


def kernel(x, token_table, pos_table):
    xp = jnp.pad(x.astype(jnp.int32), ((0, 0), (0, LPAD - MAXLEN)))
    return _emb(xp, token_table, pos_table)


# speedup vs baseline: 1.5051x; 1.0278x over previous
import jax
import jax.numpy as jnp
from jax import lax
from jax.experimental import pallas as pl
from jax.experimental.pallas import tpu as pltpu
from jax.experimental.pallas import tpu_sc as plsc

MAXLEN = 200
LPAD = 256
EMBED = 64
BATCH = 1024
NW = 32
ROWS_PER_W = BATCH // NW
NBUF = 3
LANES = 16
CPR = EMBED // LANES


def _body(x_hbm, tok_hbm, pos_hbm, out_hbm,
          xidx_v, pos_v, spmem_x, gbufs, sidxs, gsems, osems, isems):
    sid = lax.axis_index("s")
    wid = sid * 2 + lax.axis_index("c")
    base = wid * ROWS_PER_W

    pltpu.sync_copy(x_hbm.at[pl.ds(base, ROWS_PER_W)], xidx_v)
    pltpu.sync_copy(xidx_v, spmem_x.at[sid])
    pltpu.sync_copy(pos_hbm, pos_v)

    def stage_idx(t):
        p = t % NBUF
        return pltpu.async_copy(spmem_x.at[sid, t], sidxs[p], isems[p])

    def fire_gathers(t):
        p = t % NBUF
        sidx = sidxs[p]
        gbuf = gbufs[p]
        sem = gsems[p]

        def one(i, carry):
            idx = sidx[i]
            pltpu.async_copy(tok_hbm.at[idx], gbuf.at[i], sem)
            return carry
        lax.fori_loop(0, MAXLEN, one, 0, unroll=8)

    def drain_gathers(t):
        p = t % NBUF
        pltpu.make_async_copy(tok_hbm.at[pl.ds(0, MAXLEN)], gbufs[p],
                              gsems[p]).wait()

    def add_pos(t):
        p = t % NBUF
        gbuf = gbufs[p]

        def row(r, carry):
            for c in range(CPR):
                s = pl.ds(c * LANES, LANES)
                gbuf[r, s] = gbuf[r, s] + pos_v[r, s]
            return carry
        lax.fori_loop(0, MAXLEN, row, 0, unroll=4)

    ih = [None] * NBUF
    oh = [None] * NBUF
    for t in range(min(NBUF, ROWS_PER_W)):
        ih[t % NBUF] = stage_idx(t)
    ih[0].wait()
    fire_gathers(0)

    for t in range(ROWS_PER_W):
        p = t % NBUF
        q = (t + 1) % NBUF
        if t + 1 < ROWS_PER_W:
            if oh[q] is not None:
                oh[q].wait()
                oh[q] = None
            ih[q].wait()
            fire_gathers(t + 1)
        if t + NBUF < ROWS_PER_W:
            ih[p] = stage_idx(t + NBUF)
        drain_gathers(t)
        add_pos(t)
        oh[p] = pltpu.async_copy(gbufs[p], out_hbm.at[base + t], osems[p])
    for h in oh:
        if h is not None:
            h.wait()


@jax.jit
def _emb(x, token_table, pos_table):
    mesh = plsc.VectorSubcoreMesh(core_axis_name="c", subcore_axis_name="s")

    def body(x_hbm, tok_hbm, pos_hbm, out_hbm,
             xidx_v, pos_v, spmem_x,
             g0, g1, g2, s0, s1, s2,
             gs0, gs1, gs2, os0, os1, os2, is0, is1, is2):
        _body(x_hbm, tok_hbm, pos_hbm, out_hbm, xidx_v, pos_v, spmem_x,
              (g0, g1, g2), (s0, s1, s2),
              (gs0, gs1, gs2), (os0, os1, os2), (is0, is1, is2))

    run = pl.kernel(
        body,
        out_type=jax.ShapeDtypeStruct((BATCH, MAXLEN, EMBED), jnp.float32),
        mesh=mesh,
        compiler_params=pltpu.CompilerParams(use_tc_tiling_on_sc=True),
        scratch_types=[
            pltpu.VMEM((ROWS_PER_W, LPAD), jnp.int32),
            pltpu.VMEM((MAXLEN, EMBED), jnp.float32),
            pltpu.VMEM_SHARED((16, ROWS_PER_W, LPAD), jnp.int32),
            pltpu.VMEM((MAXLEN, EMBED), jnp.float32),
            pltpu.VMEM((MAXLEN, EMBED), jnp.float32),
            pltpu.VMEM((MAXLEN, EMBED), jnp.float32),
            pltpu.SMEM((LPAD,), jnp.int32),
            pltpu.SMEM((LPAD,), jnp.int32),
            pltpu.SMEM((LPAD,), jnp.int32),
            pltpu.SemaphoreType.DMA,
            pltpu.SemaphoreType.DMA,
            pltpu.SemaphoreType.DMA,
            pltpu.SemaphoreType.DMA,
            pltpu.SemaphoreType.DMA,
            pltpu.SemaphoreType.DMA,
            pltpu.SemaphoreType.DMA,
            pltpu.SemaphoreType.DMA,
            pltpu.SemaphoreType.DMA,
        ],
    )
    return run(x, token_table, pos_table)


def kernel(x, token_table, pos_table):
    xp = jnp.pad(x.astype(jnp.int32), ((0, 0), (0, LPAD - MAXLEN)))
    return _emb(xp, token_table, pos_table)
